# baseline (speedup 1.0000x reference)
import jax
import jax.numpy as jnp
from jax.experimental import pallas as pl
from jax.experimental.pallas import tpu as pltpu

N_IN = 3072
N_HID = 2048
N_OUT = 100
BATCH = 4096
BM = 256


def _mlp_body(x_ref, w0_hbm, b0_ref, w1_hbm, b1_ref, w2_hbm, b2_ref,
              o_ref, w0_v, w1_v, w2_v, sem):
    @pl.when(pl.program_id(0) == 0)
    def _load_weights():
        c0 = pltpu.make_async_copy(w0_hbm, w0_v, sem)
        c1 = pltpu.make_async_copy(w1_hbm, w1_v, sem)
        c2 = pltpu.make_async_copy(w2_hbm, w2_v, sem)
        c0.start()
        c1.start()
        c2.start()
        c0.wait()
        c1.wait()
        c2.wait()

    h = jnp.dot(x_ref[...], w0_v[...], preferred_element_type=jnp.float32)
    h = jnp.maximum(h + b0_ref[...], 0.0)
    h = jnp.dot(h, w1_v[...], preferred_element_type=jnp.float32)
    h = jnp.maximum(h + b1_ref[...], 0.0)
    o_ref[...] = (
        jnp.dot(h, w2_v[...], preferred_element_type=jnp.float32) + b2_ref[...]
    )


def kernel(x, W0, b0, W1, b1, W2, b2):
    b0r = b0.reshape(1, N_HID)
    b1r = b1.reshape(1, N_HID)
    b2r = b2.reshape(1, N_OUT)
    grid = (BATCH // BM,)
    return pl.pallas_call(
        _mlp_body,
        grid=grid,
        in_specs=[
            pl.BlockSpec((BM, N_IN), lambda i: (i, 0)),
            pl.BlockSpec(memory_space=pl.ANY),
            pl.BlockSpec((1, N_HID), lambda i: (0, 0)),
            pl.BlockSpec(memory_space=pl.ANY),
            pl.BlockSpec((1, N_HID), lambda i: (0, 0)),
            pl.BlockSpec(memory_space=pl.ANY),
            pl.BlockSpec((1, N_OUT), lambda i: (0, 0)),
        ],
        out_specs=pl.BlockSpec((BM, N_OUT), lambda i: (i, 0)),
        out_shape=jax.ShapeDtypeStruct((BATCH, N_OUT), jnp.float32),
        scratch_shapes=[
            pltpu.VMEM((N_IN, N_HID), jnp.float32),
            pltpu.VMEM((N_HID, N_HID), jnp.float32),
            pltpu.VMEM((N_HID, N_OUT), jnp.float32),
            pltpu.SemaphoreType.DMA,
        ],
        compiler_params=pltpu.CompilerParams(
            dimension_semantics=("arbitrary",),
        ),
    )(x, W0, b0r, W1, b1r, W2, b2r)
